# baseline (speedup 1.0000x reference)
import functools

import jax
import jax.numpy as jnp
from jax import lax
from jax.experimental import pallas as pl
from jax.experimental.pallas import tpu as pltpu
from jax.experimental.pallas import tpu_sc as plsc

T = 2048
DIM = 2049
S = 1408
INTER = 1409
E = 64
K = 2

DPAD = 2176
OUT = 2048
YPAD = 2176
B = 64
NROWS_R = 4096 + E * B
NBLK_R = NROWS_R // B
BS = 256

NC = 2
NS = 16
NW = NC * NS
GCH = 16
CCH = 16

_f32 = jnp.float32
_i32 = jnp.int32


def _gate(x, gate_w):
    def body(x_ref, gw_ref, tw_ref, ti_ref):
        logits = jnp.dot(x_ref[...], gw_ref[...], preferred_element_type=_f32)
        s = jax.nn.softmax(logits, axis=-1)
        iota = lax.broadcasted_iota(_i32, (T, E), 1)
        m1 = jnp.max(s, axis=-1, keepdims=True)
        i1 = jnp.min(jnp.where(s >= m1, iota, E), axis=-1, keepdims=True)
        s2 = jnp.where(iota == i1, -1e30, s)
        m2 = jnp.max(s2, axis=-1, keepdims=True)
        i2 = jnp.min(jnp.where(s2 >= m2, iota, E), axis=-1, keepdims=True)
        tw_ref[...] = jnp.concatenate([m1, m2], axis=-1)
        ti_ref[...] = jnp.concatenate([i1, i2], axis=-1)

    return pl.pallas_call(
        body,
        out_shape=[jax.ShapeDtypeStruct((T, K), _f32),
                   jax.ShapeDtypeStruct((T, K), _i32)],
    )(x, gate_w)


def _routing(ti, tw):
    eid = ti.reshape(-1).astype(_i32)
    wts = tw.reshape(-1)
    order = jnp.argsort(eid, stable=True).astype(_i32)
    sorted_eid = eid[order]
    counts = jnp.zeros((E,), _i32).at[eid].add(1)
    blocks_e = (counts + B - 1) // B
    blk_start = jnp.concatenate(
        [jnp.zeros((1,), _i32), jnp.cumsum(blocks_e)[:-1].astype(_i32)])
    row_start = blk_start * B
    csum_counts = jnp.concatenate(
        [jnp.zeros((1,), _i32), jnp.cumsum(counts)[:-1].astype(_i32)])
    ranks = jnp.arange(T * K, dtype=_i32) - csum_counts[sorted_eid]
    dst_row = row_start[sorted_eid] + ranks
    src = jnp.zeros((NROWS_R,), _i32).at[dst_row].set(order // K)
    wvals = jnp.zeros((NROWS_R,), _f32).at[dst_row].set(wts[order])
    pos_flat = jnp.zeros((T * K,), _i32).at[order].set(dst_row)
    bi = jnp.arange(NBLK_R, dtype=_i32)
    blk_eid = jnp.clip(
        jnp.searchsorted(blk_start, bi, side="right").astype(_i32) - 1,
        0, E - 1)
    return dict(
        src=src,
        wrow2d=jnp.broadcast_to(wvals[:, None], (NROWS_R, 128)),
        pos0=pos_flat[0::2],
        pos1=pos_flat[1::2],
        blk_eid=blk_eid,
    )


def _mlp_stage1(xb, w1, w3):
    s1 = jnp.dot(xb, w1, preferred_element_type=_f32)
    s3 = jnp.dot(xb, w3, preferred_element_type=_f32)
    return jax.nn.silu(s1) * s3


def _mlp_stage2(sp, w2):
    t = jnp.sqrt(jnp.clip(
        jnp.sum(sp * sp, axis=-1, keepdims=True) + 1.0, 1e-6, None))
    hfull = jnp.concatenate([t, sp], axis=-1)
    s2 = jnp.dot(hfull, w2, preferred_element_type=_f32)
    t2 = jnp.sqrt(jnp.clip(
        jnp.sum(s2 * s2, axis=-1, keepdims=True) + 1.0, 1e-6, None))
    return s2, t2


def _expert_a(blk_eid, xs, w1v, w3v):
    def body(eids_ref, x_ref, w1_ref, w3_ref, h_ref):
        h_ref[...] = _mlp_stage1(x_ref[...][:, :DIM], w1_ref[...], w3_ref[...])

    return pl.pallas_call(
        body,
        grid_spec=pltpu.PrefetchScalarGridSpec(
            num_scalar_prefetch=1,
            grid=(NBLK_R,),
            in_specs=[
                pl.BlockSpec((B, DPAD), lambda b, eids: (b, 0)),
                pl.BlockSpec((DIM, S), lambda b, eids: (0, eids[b])),
                pl.BlockSpec((DIM, S), lambda b, eids: (0, eids[b])),
            ],
            out_specs=pl.BlockSpec((B, S), lambda b, eids: (b, 0)),
        ),
        out_shape=jax.ShapeDtypeStruct((NROWS_R, S), _f32),
    )(blk_eid, xs, w1v, w3v)


def _expert_b(blk_eid, h, w2v, wrow2d):
    def body(eids_ref, h_ref, w2_ref, wr_ref, y_ref):
        s2, t2 = _mlp_stage2(h_ref[...], w2_ref[...])
        w = wr_ref[:, 0:1]
        z = jnp.zeros((B, YPAD - OUT - 1), _f32)
        y_ref[...] = jnp.concatenate([w * s2, w * t2, z], axis=-1)

    return pl.pallas_call(
        body,
        grid_spec=pltpu.PrefetchScalarGridSpec(
            num_scalar_prefetch=1,
            grid=(NBLK_R,),
            in_specs=[
                pl.BlockSpec((B, S), lambda b, eids: (b, 0)),
                pl.BlockSpec((INTER, OUT), lambda b, eids: (0, eids[b])),
                pl.BlockSpec((B, 128), lambda b, eids: (b, 0)),
            ],
            out_specs=pl.BlockSpec((B, YPAD), lambda b, eids: (b, 0)),
        ),
        out_shape=jax.ShapeDtypeStruct((NROWS_R, YPAD), _f32),
    )(blk_eid, h, w2v, wrow2d)


def _shared_a(x, sw1, sw3):
    def body(x_ref, w1_ref, w3_ref, h_ref):
        h_ref[...] = _mlp_stage1(x_ref[...], w1_ref[...], w3_ref[...])

    return pl.pallas_call(
        body,
        grid=(T // BS,),
        in_specs=[
            pl.BlockSpec((BS, DIM), lambda b: (b, 0)),
            pl.BlockSpec((DIM, S), lambda b: (0, 0)),
            pl.BlockSpec((DIM, S), lambda b: (0, 0)),
        ],
        out_specs=pl.BlockSpec((BS, S), lambda b: (b, 0)),
        out_shape=jax.ShapeDtypeStruct((T, S), _f32),
    )(x, sw1, sw3)


def _shared_b(h, sw2):
    def body(h_ref, w2_ref, z_ref):
        s2, t2 = _mlp_stage2(h_ref[...], w2_ref[...])
        z = jnp.zeros((BS, YPAD - OUT - 1), _f32)
        z_ref[...] = jnp.concatenate([s2, t2, z], axis=-1)

    return pl.pallas_call(
        body,
        grid=(T // BS,),
        in_specs=[
            pl.BlockSpec((BS, S), lambda b: (b, 0)),
            pl.BlockSpec((INTER, OUT), lambda b: (0, 0)),
        ],
        out_specs=pl.BlockSpec((BS, YPAD), lambda b: (b, 0)),
        out_shape=jax.ShapeDtypeStruct((T, YPAD), _f32),
    )(h, sw2)


def _sc_mesh():
    return plsc.VectorSubcoreMesh(
        core_axis_name="c", subcore_axis_name="s", num_cores=NC)


def _sc_gather(src, xpad):
    nch = (NROWS_R // NW) // GCH

    @functools.partial(
        pl.kernel,
        out_type=jax.ShapeDtypeStruct((NROWS_R, DPAD), _f32),
        mesh=_sc_mesh(),
        scratch_types=[
            pltpu.VMEM((GCH,), _i32),
            pltpu.VMEM((GCH,), _i32),
            pltpu.VMEM((GCH, DPAD), _f32),
            pltpu.VMEM((GCH, DPAD), _f32),
            pltpu.SemaphoreType.DMA,
            pltpu.SemaphoreType.DMA,
            pltpu.SemaphoreType.DMA,
            pltpu.SemaphoreType.DMA,
        ],
    )
    def k(src_hbm, x_hbm, out_hbm, idx0, idx1, rows0, rows1, g0, g1, s0, s1):
        wid = lax.axis_index("s") * NC + lax.axis_index("c")
        base = wid * (NROWS_R // NW)
        idxs, rows, gsem, ssem = [idx0, idx1], [rows0, rows1], [g0, g1], [s0, s1]
        gh = [None, None]
        sh = [None, None]

        pltpu.sync_copy(src_hbm.at[pl.ds(base, GCH)], idxs[0])
        gh[0] = pltpu.async_copy(x_hbm.at[idxs[0]], rows[0], gsem[0])
        for i in range(1, nch):
            b, pb = i % 2, (i - 1) % 2
            if i >= 2:
                sh[b].wait()
            pltpu.sync_copy(src_hbm.at[pl.ds(base + i * GCH, GCH)], idxs[b])
            gh[b] = pltpu.async_copy(x_hbm.at[idxs[b]], rows[b], gsem[b])
            gh[pb].wait()
            sh[pb] = pltpu.async_copy(
                rows[pb], out_hbm.at[pl.ds(base + (i - 1) * GCH, GCH)],
                ssem[pb])
        lb = (nch - 1) % 2
        gh[lb].wait()
        sh[lb] = pltpu.async_copy(
            rows[lb], out_hbm.at[pl.ds(base + (nch - 1) * GCH, GCH)], ssem[lb])
        sh[lb].wait()
        if nch >= 2:
            sh[1 - lb].wait()

    return k(src, xpad)


def _sc_combine(z, ys, pos0, pos1):
    @functools.partial(
        pl.kernel,
        out_type=jax.ShapeDtypeStruct((T, YPAD), _f32),
        mesh=_sc_mesh(),
        scratch_types=[
            pltpu.VMEM((CCH,), _i32),
            pltpu.VMEM((CCH,), _i32),
            pltpu.VMEM((CCH, YPAD), _f32),
            pltpu.VMEM((CCH, YPAD), _f32),
            pltpu.VMEM((CCH, YPAD), _f32),
            pltpu.SemaphoreType.DMA,
        ],
    )
    def k(z_hbm, ys_hbm, pos0_hbm, pos1_hbm, out_hbm,
          i0_v, i1_v, rz_v, r0_v, r1_v, sem):
        wid = lax.axis_index("s") * NC + lax.axis_index("c")
        base = wid * (T // NW)

        def chunk(i, carry):
            off = base + i * CCH
            pltpu.sync_copy(pos0_hbm.at[pl.ds(off, CCH)], i0_v)
            pltpu.sync_copy(pos1_hbm.at[pl.ds(off, CCH)], i1_v)
            pltpu.sync_copy(z_hbm.at[pl.ds(off, CCH)], rz_v)
            pltpu.async_copy(ys_hbm.at[i0_v], r0_v, sem).wait()
            pltpu.async_copy(ys_hbm.at[i1_v], r1_v, sem).wait()

            def row(rr, c2):
                def col(cc, c3):
                    sl = pl.ds(cc * 16, 16)
                    rz_v[rr, sl] = rz_v[rr, sl] + r0_v[rr, sl] + r1_v[rr, sl]
                    return c3
                lax.fori_loop(0, YPAD // 16, col, 0)
                return c2

            lax.fori_loop(0, CCH, row, 0)
            pltpu.sync_copy(rz_v, out_hbm.at[pl.ds(off, CCH)])
            return carry

        lax.fori_loop(0, (T // NW) // CCH, chunk, 0)

    return k(z, ys, pos0, pos1)


def kernel(x, gate_w, w1, w2, w3, sw1, sw2, sw3):
    tw, ti = _gate(x, gate_w)
    r = _routing(ti, tw)
    xpad = jnp.pad(x, ((0, 0), (0, DPAD - DIM)))
    xs = _sc_gather(r["src"], xpad)
    w1v = jnp.transpose(w1, (1, 0, 2)).reshape(DIM, E * S)
    w3v = jnp.transpose(w3, (1, 0, 2)).reshape(DIM, E * S)
    w2v = jnp.transpose(w2, (1, 0, 2)).reshape(INTER, E * OUT)
    h = _expert_a(r["blk_eid"], xs, w1v, w3v)
    ys = _expert_b(r["blk_eid"], h, w2v, r["wrow2d"])
    hs = _shared_a(x, sw1, sw3)
    z = _shared_b(hs, sw2)
    ypad = _sc_combine(z, ys, r["pos0"], r["pos1"])
    return jnp.concatenate([ypad[:, OUT:OUT + 1], ypad[:, :OUT]], axis=1)


# speedup vs baseline: 1.3809x; 1.3809x over previous
import functools

import jax
import jax.numpy as jnp
from jax import lax
from jax.experimental import pallas as pl
from jax.experimental.pallas import tpu as pltpu
from jax.experimental.pallas import tpu_sc as plsc

T = 2048
DIM = 2049
S = 1408
INTER = 1409
E = 64
K = 2

DPAD = 2176
OUT = 2048
YPAD = 2176
B = 64
NROWS_R = 4096 + E * B
NBLK_R = NROWS_R // B
BS = 256

NC = 2
NS = 16
NW = NC * NS
GCH = 16
CCH = 16

_f32 = jnp.float32
_i32 = jnp.int32


def _gate(x, gate_w):
    def body(x_ref, gw_ref, tw_ref, ti_ref):
        logits = jnp.dot(x_ref[...], gw_ref[...], preferred_element_type=_f32)
        s = jax.nn.softmax(logits, axis=-1)
        iota = lax.broadcasted_iota(_i32, (T, E), 1)
        m1 = jnp.max(s, axis=-1, keepdims=True)
        i1 = jnp.min(jnp.where(s >= m1, iota, E), axis=-1, keepdims=True)
        s2 = jnp.where(iota == i1, -1e30, s)
        m2 = jnp.max(s2, axis=-1, keepdims=True)
        i2 = jnp.min(jnp.where(s2 >= m2, iota, E), axis=-1, keepdims=True)
        tw_ref[...] = jnp.concatenate([m1, m2], axis=-1)
        ti_ref[...] = jnp.concatenate([i1, i2], axis=-1)

    return pl.pallas_call(
        body,
        out_shape=[jax.ShapeDtypeStruct((T, K), _f32),
                   jax.ShapeDtypeStruct((T, K), _i32)],
    )(x, gate_w)


def _routing(ti, tw):
    eid = ti.reshape(-1).astype(_i32)
    wts = tw.reshape(-1)
    order = jnp.argsort(eid, stable=True).astype(_i32)
    sorted_eid = eid[order]
    counts = jnp.zeros((E,), _i32).at[eid].add(1)
    blocks_e = (counts + B - 1) // B
    blk_start = jnp.concatenate(
        [jnp.zeros((1,), _i32), jnp.cumsum(blocks_e)[:-1].astype(_i32)])
    row_start = blk_start * B
    csum_counts = jnp.concatenate(
        [jnp.zeros((1,), _i32), jnp.cumsum(counts)[:-1].astype(_i32)])
    ranks = jnp.arange(T * K, dtype=_i32) - csum_counts[sorted_eid]
    dst_row = row_start[sorted_eid] + ranks
    src = jnp.zeros((NROWS_R,), _i32).at[dst_row].set(order // K)
    wvals = jnp.zeros((NROWS_R,), _f32).at[dst_row].set(wts[order])
    pos_flat = jnp.zeros((T * K,), _i32).at[order].set(dst_row)
    bi = jnp.arange(NBLK_R, dtype=_i32)
    blk_eid = jnp.clip(
        jnp.searchsorted(blk_start, bi, side="right").astype(_i32) - 1,
        0, E - 1)
    return dict(
        src=src,
        wrow2d=jnp.broadcast_to(wvals[:, None], (NROWS_R, 128)),
        pos0=pos_flat[0::2],
        pos1=pos_flat[1::2],
        blk_eid=blk_eid,
    )


def _mlp_stage1(xb, w1, w3):
    xb = xb.astype(w1.dtype)
    s1 = jnp.dot(xb, w1, preferred_element_type=_f32)
    s3 = jnp.dot(xb, w3, preferred_element_type=_f32)
    return jax.nn.silu(s1) * s3


def _mlp_stage2(sp, w2):
    t = jnp.sqrt(jnp.clip(
        jnp.sum(sp * sp, axis=-1, keepdims=True) + 1.0, 1e-6, None))
    hfull = jnp.concatenate([t, sp], axis=-1)
    s2 = jnp.dot(hfull.astype(w2.dtype), w2, preferred_element_type=_f32)
    t2 = jnp.sqrt(jnp.clip(
        jnp.sum(s2 * s2, axis=-1, keepdims=True) + 1.0, 1e-6, None))
    return s2, t2


def _expert_a(blk_eid, xs, w1b, w3b):
    def body(eids_ref, x_ref, w1_ref, w3_ref, h_ref):
        h_ref[...] = _mlp_stage1(x_ref[...][:, :DIM], w1_ref[0], w3_ref[0])

    return pl.pallas_call(
        body,
        grid_spec=pltpu.PrefetchScalarGridSpec(
            num_scalar_prefetch=1,
            grid=(NBLK_R,),
            in_specs=[
                pl.BlockSpec((B, DPAD), lambda b, eids: (b, 0)),
                pl.BlockSpec((1, DIM, S), lambda b, eids: (eids[b], 0, 0)),
                pl.BlockSpec((1, DIM, S), lambda b, eids: (eids[b], 0, 0)),
            ],
            out_specs=pl.BlockSpec((B, S), lambda b, eids: (b, 0)),
        ),
        out_shape=jax.ShapeDtypeStruct((NROWS_R, S), _f32),
    )(blk_eid, xs, w1b, w3b)


def _expert_b(blk_eid, h, w2b, wrow2d):
    def body(eids_ref, h_ref, w2_ref, wr_ref, y_ref):
        s2, t2 = _mlp_stage2(h_ref[...], w2_ref[0])
        w = wr_ref[:, 0:1]
        z = jnp.zeros((B, YPAD - OUT - 1), _f32)
        y_ref[...] = jnp.concatenate([w * s2, w * t2, z], axis=-1)

    return pl.pallas_call(
        body,
        grid_spec=pltpu.PrefetchScalarGridSpec(
            num_scalar_prefetch=1,
            grid=(NBLK_R,),
            in_specs=[
                pl.BlockSpec((B, S), lambda b, eids: (b, 0)),
                pl.BlockSpec((1, INTER, OUT), lambda b, eids: (eids[b], 0, 0)),
                pl.BlockSpec((B, 128), lambda b, eids: (b, 0)),
            ],
            out_specs=pl.BlockSpec((B, YPAD), lambda b, eids: (b, 0)),
        ),
        out_shape=jax.ShapeDtypeStruct((NROWS_R, YPAD), _f32),
    )(blk_eid, h, w2b, wrow2d)


def _shared_a(x, sw1, sw3):
    def body(x_ref, w1_ref, w3_ref, h_ref):
        h_ref[...] = _mlp_stage1(x_ref[...], w1_ref[...], w3_ref[...])

    return pl.pallas_call(
        body,
        grid=(T // BS,),
        in_specs=[
            pl.BlockSpec((BS, DIM), lambda b: (b, 0)),
            pl.BlockSpec((DIM, S), lambda b: (0, 0)),
            pl.BlockSpec((DIM, S), lambda b: (0, 0)),
        ],
        out_specs=pl.BlockSpec((BS, S), lambda b: (b, 0)),
        out_shape=jax.ShapeDtypeStruct((T, S), _f32),
    )(x, sw1, sw3)


def _shared_b(h, sw2):
    def body(h_ref, w2_ref, z_ref):
        s2, t2 = _mlp_stage2(h_ref[...], w2_ref[...])
        z = jnp.zeros((BS, YPAD - OUT - 1), _f32)
        z_ref[...] = jnp.concatenate([s2, t2, z], axis=-1)

    return pl.pallas_call(
        body,
        grid=(T // BS,),
        in_specs=[
            pl.BlockSpec((BS, S), lambda b: (b, 0)),
            pl.BlockSpec((INTER, OUT), lambda b: (0, 0)),
        ],
        out_specs=pl.BlockSpec((BS, YPAD), lambda b: (b, 0)),
        out_shape=jax.ShapeDtypeStruct((T, YPAD), _f32),
    )(h, sw2)


def _sc_mesh():
    return plsc.VectorSubcoreMesh(
        core_axis_name="c", subcore_axis_name="s", num_cores=NC)


def _sc_gather(src, xpad):
    nch = (NROWS_R // NW) // GCH

    @functools.partial(
        pl.kernel,
        out_type=jax.ShapeDtypeStruct((NROWS_R, DPAD), _f32),
        mesh=_sc_mesh(),
        scratch_types=[
            pltpu.VMEM((GCH,), _i32),
            pltpu.VMEM((GCH,), _i32),
            pltpu.VMEM((GCH, DPAD), _f32),
            pltpu.VMEM((GCH, DPAD), _f32),
            pltpu.SemaphoreType.DMA,
            pltpu.SemaphoreType.DMA,
            pltpu.SemaphoreType.DMA,
            pltpu.SemaphoreType.DMA,
        ],
    )
    def k(src_hbm, x_hbm, out_hbm, idx0, idx1, rows0, rows1, g0, g1, s0, s1):
        wid = lax.axis_index("s") * NC + lax.axis_index("c")
        base = wid * (NROWS_R // NW)
        idxs, rows, gsem, ssem = [idx0, idx1], [rows0, rows1], [g0, g1], [s0, s1]
        gh = [None, None]
        sh = [None, None]

        pltpu.sync_copy(src_hbm.at[pl.ds(base, GCH)], idxs[0])
        gh[0] = pltpu.async_copy(x_hbm.at[idxs[0]], rows[0], gsem[0])
        for i in range(1, nch):
            b, pb = i % 2, (i - 1) % 2
            if i >= 2:
                sh[b].wait()
            pltpu.sync_copy(src_hbm.at[pl.ds(base + i * GCH, GCH)], idxs[b])
            gh[b] = pltpu.async_copy(x_hbm.at[idxs[b]], rows[b], gsem[b])
            gh[pb].wait()
            sh[pb] = pltpu.async_copy(
                rows[pb], out_hbm.at[pl.ds(base + (i - 1) * GCH, GCH)],
                ssem[pb])
        lb = (nch - 1) % 2
        gh[lb].wait()
        sh[lb] = pltpu.async_copy(
            rows[lb], out_hbm.at[pl.ds(base + (nch - 1) * GCH, GCH)], ssem[lb])
        sh[lb].wait()
        if nch >= 2:
            sh[1 - lb].wait()

    return k(src, xpad)


def _sc_combine(z, ys, pos0, pos1):
    @functools.partial(
        pl.kernel,
        out_type=jax.ShapeDtypeStruct((T, YPAD), _f32),
        mesh=_sc_mesh(),
        scratch_types=[
            pltpu.VMEM((CCH,), _i32),
            pltpu.VMEM((CCH,), _i32),
            pltpu.VMEM((CCH, YPAD), _f32),
            pltpu.VMEM((CCH, YPAD), _f32),
            pltpu.VMEM((CCH, YPAD), _f32),
            pltpu.SemaphoreType.DMA,
        ],
    )
    def k(z_hbm, ys_hbm, pos0_hbm, pos1_hbm, out_hbm,
          i0_v, i1_v, rz_v, r0_v, r1_v, sem):
        wid = lax.axis_index("s") * NC + lax.axis_index("c")
        base = wid * (T // NW)

        def chunk(i, carry):
            off = base + i * CCH
            pltpu.sync_copy(pos0_hbm.at[pl.ds(off, CCH)], i0_v)
            pltpu.sync_copy(pos1_hbm.at[pl.ds(off, CCH)], i1_v)
            pltpu.sync_copy(z_hbm.at[pl.ds(off, CCH)], rz_v)
            pltpu.async_copy(ys_hbm.at[i0_v], r0_v, sem).wait()
            pltpu.async_copy(ys_hbm.at[i1_v], r1_v, sem).wait()

            def row(rr, c2):
                def col(cc, c3):
                    sl = pl.ds(cc * 16, 16)
                    rz_v[rr, sl] = rz_v[rr, sl] + r0_v[rr, sl] + r1_v[rr, sl]
                    return c3
                lax.fori_loop(0, YPAD // 16, col, 0)
                return c2

            lax.fori_loop(0, CCH, row, 0)
            pltpu.sync_copy(rz_v, out_hbm.at[pl.ds(off, CCH)])
            return carry

        lax.fori_loop(0, (T // NW) // CCH, chunk, 0)

    return k(z, ys, pos0, pos1)


def kernel(x, gate_w, w1, w2, w3, sw1, sw2, sw3):
    tw, ti = _gate(x, gate_w)
    r = _routing(ti, tw)
    xpad = jnp.pad(x, ((0, 0), (0, DPAD - DIM)))
    xs = _sc_gather(r["src"], xpad)
    w1b = w1.astype(jnp.bfloat16)
    w3b = w3.astype(jnp.bfloat16)
    w2b = w2.astype(jnp.bfloat16)
    h = _expert_a(r["blk_eid"], xs, w1b, w3b)
    ys = _expert_b(r["blk_eid"], h, w2b, r["wrow2d"])
    hs = _shared_a(x, sw1, sw3)
    z = _shared_b(hs, sw2)
    ypad = _sc_combine(z, ys, r["pos0"], r["pos1"])
    return jnp.concatenate([ypad[:, OUT:OUT + 1], ypad[:, :OUT]], axis=1)


# speedup vs baseline: 1.7122x; 1.2399x over previous
import functools

import jax
import jax.numpy as jnp
from jax import lax
from jax.experimental import pallas as pl
from jax.experimental.pallas import tpu as pltpu
from jax.experimental.pallas import tpu_sc as plsc

T = 2048
DIM = 2049
S = 1408
INTER = 1409
E = 64
K = 2

DPAD = 2176
OUT = 2048
YPAD = 2176
B = 64
NROWS_R = 4096 + E * B
NBLK_R = NROWS_R // B
BS = 256

NC = 2
NS = 16
NW = NC * NS
GCH = 16
CCH = 16

_f32 = jnp.float32
_i32 = jnp.int32


def _gate(x, gate_w):
    def body(x_ref, gw_ref, tw_ref, ti_ref):
        logits = jnp.dot(x_ref[...], gw_ref[...], preferred_element_type=_f32)
        s = jax.nn.softmax(logits, axis=-1)
        iota = lax.broadcasted_iota(_i32, (T, E), 1)
        m1 = jnp.max(s, axis=-1, keepdims=True)
        i1 = jnp.min(jnp.where(s >= m1, iota, E), axis=-1, keepdims=True)
        s2 = jnp.where(iota == i1, -1e30, s)
        m2 = jnp.max(s2, axis=-1, keepdims=True)
        i2 = jnp.min(jnp.where(s2 >= m2, iota, E), axis=-1, keepdims=True)
        tw_ref[...] = jnp.concatenate([m1, m2], axis=-1)
        ti_ref[...] = jnp.concatenate([i1, i2], axis=-1)

    return pl.pallas_call(
        body,
        out_shape=[jax.ShapeDtypeStruct((T, K), _f32),
                   jax.ShapeDtypeStruct((T, K), _i32)],
    )(x, gate_w)


def _routing(ti, tw):
    eid = ti.reshape(-1).astype(_i32)
    wts = tw.reshape(-1)
    order = jnp.argsort(eid, stable=True).astype(_i32)
    sorted_eid = eid[order]
    counts = jnp.zeros((E,), _i32).at[eid].add(1)
    blocks_e = (counts + B - 1) // B
    blk_start = jnp.concatenate(
        [jnp.zeros((1,), _i32), jnp.cumsum(blocks_e)[:-1].astype(_i32)])
    row_start = blk_start * B
    csum_counts = jnp.concatenate(
        [jnp.zeros((1,), _i32), jnp.cumsum(counts)[:-1].astype(_i32)])
    ranks = jnp.arange(T * K, dtype=_i32) - csum_counts[sorted_eid]
    dst_row = row_start[sorted_eid] + ranks
    src = jnp.zeros((NROWS_R,), _i32).at[dst_row].set(order // K)
    wvals = jnp.zeros((NROWS_R,), _f32).at[dst_row].set(wts[order])
    pos_flat = jnp.zeros((T * K,), _i32).at[order].set(dst_row)
    bi = jnp.arange(NBLK_R, dtype=_i32)
    blk_eid = jnp.clip(
        jnp.searchsorted(blk_start, bi, side="right").astype(_i32) - 1,
        0, E - 1)
    return dict(
        src=src,
        wrow2d=jnp.broadcast_to(wvals[:, None], (NROWS_R, 128)),
        pos0=pos_flat[0::2],
        pos1=pos_flat[1::2],
        blk_eid=blk_eid,
    )


def _mlp_stage1(xb, w1, w3):
    xb = xb.astype(w1.dtype)
    s1 = jnp.dot(xb, w1, preferred_element_type=_f32)
    s3 = jnp.dot(xb, w3, preferred_element_type=_f32)
    return jax.nn.silu(s1) * s3


def _mlp_stage2(sp, w2):
    t = jnp.sqrt(jnp.clip(
        jnp.sum(sp * sp, axis=-1, keepdims=True) + 1.0, 1e-6, None))
    hfull = jnp.concatenate([t, sp], axis=-1)
    s2 = jnp.dot(hfull.astype(w2.dtype), w2, preferred_element_type=_f32)
    t2 = jnp.sqrt(jnp.clip(
        jnp.sum(s2 * s2, axis=-1, keepdims=True) + 1.0, 1e-6, None))
    return s2, t2


def _wprep(w, d, s):
    wv = jnp.transpose(w, (1, 0, 2))
    nblk = (d + 7) // 8

    def body(in_ref, out_ref):
        out_ref[...] = jnp.transpose(in_ref[...], (1, 0, 2)).astype(
            jnp.bfloat16)

    return pl.pallas_call(
        body,
        grid=(nblk,),
        in_specs=[pl.BlockSpec((8, E, s), lambda b: (b, 0, 0))],
        out_specs=pl.BlockSpec((E, 8, s), lambda b: (0, b, 0)),
        out_shape=jax.ShapeDtypeStruct((E, d, s), jnp.bfloat16),
    )(wv)


def _expert_a(blk_eid, xs, w1b, w3b):
    def body(eids_ref, x_ref, w1_ref, w3_ref, h_ref):
        h_ref[...] = _mlp_stage1(x_ref[...][:, :DIM], w1_ref[0], w3_ref[0])

    return pl.pallas_call(
        body,
        grid_spec=pltpu.PrefetchScalarGridSpec(
            num_scalar_prefetch=1,
            grid=(NBLK_R,),
            in_specs=[
                pl.BlockSpec((B, DPAD), lambda b, eids: (b, 0)),
                pl.BlockSpec((1, DIM, S), lambda b, eids: (eids[b], 0, 0)),
                pl.BlockSpec((1, DIM, S), lambda b, eids: (eids[b], 0, 0)),
            ],
            out_specs=pl.BlockSpec((B, S), lambda b, eids: (b, 0)),
        ),
        out_shape=jax.ShapeDtypeStruct((NROWS_R, S), _f32),
    )(blk_eid, xs, w1b, w3b)


def _expert_b(blk_eid, h, w2b, wrow2d):
    def body(eids_ref, h_ref, w2_ref, wr_ref, y_ref):
        s2, t2 = _mlp_stage2(h_ref[...], w2_ref[0])
        w = wr_ref[:, 0:1]
        z = jnp.zeros((B, YPAD - OUT - 1), _f32)
        y_ref[...] = jnp.concatenate([w * s2, w * t2, z], axis=-1)

    return pl.pallas_call(
        body,
        grid_spec=pltpu.PrefetchScalarGridSpec(
            num_scalar_prefetch=1,
            grid=(NBLK_R,),
            in_specs=[
                pl.BlockSpec((B, S), lambda b, eids: (b, 0)),
                pl.BlockSpec((1, INTER, OUT), lambda b, eids: (eids[b], 0, 0)),
                pl.BlockSpec((B, 128), lambda b, eids: (b, 0)),
            ],
            out_specs=pl.BlockSpec((B, YPAD), lambda b, eids: (b, 0)),
        ),
        out_shape=jax.ShapeDtypeStruct((NROWS_R, YPAD), _f32),
    )(blk_eid, h, w2b, wrow2d)


def _shared_a(x, sw1, sw3):
    def body(x_ref, w1_ref, w3_ref, h_ref):
        h_ref[...] = _mlp_stage1(x_ref[...], w1_ref[...], w3_ref[...])

    return pl.pallas_call(
        body,
        grid=(T // BS,),
        in_specs=[
            pl.BlockSpec((BS, DIM), lambda b: (b, 0)),
            pl.BlockSpec((DIM, S), lambda b: (0, 0)),
            pl.BlockSpec((DIM, S), lambda b: (0, 0)),
        ],
        out_specs=pl.BlockSpec((BS, S), lambda b: (b, 0)),
        out_shape=jax.ShapeDtypeStruct((T, S), _f32),
    )(x, sw1, sw3)


def _shared_b(h, sw2):
    def body(h_ref, w2_ref, z_ref):
        s2, t2 = _mlp_stage2(h_ref[...], w2_ref[...])
        z = jnp.zeros((BS, YPAD - OUT - 1), _f32)
        z_ref[...] = jnp.concatenate([s2, t2, z], axis=-1)

    return pl.pallas_call(
        body,
        grid=(T // BS,),
        in_specs=[
            pl.BlockSpec((BS, S), lambda b: (b, 0)),
            pl.BlockSpec((INTER, OUT), lambda b: (0, 0)),
        ],
        out_specs=pl.BlockSpec((BS, YPAD), lambda b: (b, 0)),
        out_shape=jax.ShapeDtypeStruct((T, YPAD), _f32),
    )(h, sw2)


def _sc_mesh():
    return plsc.VectorSubcoreMesh(
        core_axis_name="c", subcore_axis_name="s", num_cores=NC)


def _sc_gather(src, xpad):
    nch = (NROWS_R // NW) // GCH

    @functools.partial(
        pl.kernel,
        out_type=jax.ShapeDtypeStruct((NROWS_R, DPAD), _f32),
        mesh=_sc_mesh(),
        scratch_types=[
            pltpu.VMEM((GCH,), _i32),
            pltpu.VMEM((GCH,), _i32),
            pltpu.VMEM((GCH, DPAD), _f32),
            pltpu.VMEM((GCH, DPAD), _f32),
            pltpu.SemaphoreType.DMA,
            pltpu.SemaphoreType.DMA,
            pltpu.SemaphoreType.DMA,
            pltpu.SemaphoreType.DMA,
        ],
    )
    def k(src_hbm, x_hbm, out_hbm, idx0, idx1, rows0, rows1, g0, g1, s0, s1):
        wid = lax.axis_index("s") * NC + lax.axis_index("c")
        base = wid * (NROWS_R // NW)
        idxs, rows, gsem, ssem = [idx0, idx1], [rows0, rows1], [g0, g1], [s0, s1]
        gh = [None, None]
        sh = [None, None]

        pltpu.sync_copy(src_hbm.at[pl.ds(base, GCH)], idxs[0])
        gh[0] = pltpu.async_copy(x_hbm.at[idxs[0]], rows[0], gsem[0])
        for i in range(1, nch):
            b, pb = i % 2, (i - 1) % 2
            if i >= 2:
                sh[b].wait()
            pltpu.sync_copy(src_hbm.at[pl.ds(base + i * GCH, GCH)], idxs[b])
            gh[b] = pltpu.async_copy(x_hbm.at[idxs[b]], rows[b], gsem[b])
            gh[pb].wait()
            sh[pb] = pltpu.async_copy(
                rows[pb], out_hbm.at[pl.ds(base + (i - 1) * GCH, GCH)],
                ssem[pb])
        lb = (nch - 1) % 2
        gh[lb].wait()
        sh[lb] = pltpu.async_copy(
            rows[lb], out_hbm.at[pl.ds(base + (nch - 1) * GCH, GCH)], ssem[lb])
        sh[lb].wait()
        if nch >= 2:
            sh[1 - lb].wait()

    return k(src, xpad)


def _sc_combine(z, ys, pos0, pos1):
    @functools.partial(
        pl.kernel,
        out_type=jax.ShapeDtypeStruct((T, YPAD), _f32),
        mesh=_sc_mesh(),
        scratch_types=[
            pltpu.VMEM((CCH,), _i32),
            pltpu.VMEM((CCH,), _i32),
            pltpu.VMEM((CCH, YPAD), _f32),
            pltpu.VMEM((CCH, YPAD), _f32),
            pltpu.VMEM((CCH, YPAD), _f32),
            pltpu.SemaphoreType.DMA,
        ],
    )
    def k(z_hbm, ys_hbm, pos0_hbm, pos1_hbm, out_hbm,
          i0_v, i1_v, rz_v, r0_v, r1_v, sem):
        wid = lax.axis_index("s") * NC + lax.axis_index("c")
        base = wid * (T // NW)

        def chunk(i, carry):
            off = base + i * CCH
            pltpu.sync_copy(pos0_hbm.at[pl.ds(off, CCH)], i0_v)
            pltpu.sync_copy(pos1_hbm.at[pl.ds(off, CCH)], i1_v)
            pltpu.sync_copy(z_hbm.at[pl.ds(off, CCH)], rz_v)
            pltpu.async_copy(ys_hbm.at[i0_v], r0_v, sem).wait()
            pltpu.async_copy(ys_hbm.at[i1_v], r1_v, sem).wait()

            def row(rr, c2):
                def col(cc, c3):
                    sl = pl.ds(cc * 16, 16)
                    rz_v[rr, sl] = rz_v[rr, sl] + r0_v[rr, sl] + r1_v[rr, sl]
                    return c3
                lax.fori_loop(0, YPAD // 16, col, 0)
                return c2

            lax.fori_loop(0, CCH, row, 0)
            pltpu.sync_copy(rz_v, out_hbm.at[pl.ds(off, CCH)])
            return carry

        lax.fori_loop(0, (T // NW) // CCH, chunk, 0)

    return k(z, ys, pos0, pos1)


def kernel(x, gate_w, w1, w2, w3, sw1, sw2, sw3):
    tw, ti = _gate(x, gate_w)
    r = _routing(ti, tw)
    xpad = jnp.pad(x, ((0, 0), (0, DPAD - DIM)))
    xs = _sc_gather(r["src"], xpad)
    w1b = _wprep(w1, DIM, S)
    w3b = _wprep(w3, DIM, S)
    w2b = _wprep(w2, INTER, OUT)
    h = _expert_a(r["blk_eid"], xs, w1b, w3b)
    ys = _expert_b(r["blk_eid"], h, w2b, r["wrow2d"])
    hs = _shared_a(x, sw1, sw3)
    z = _shared_b(hs, sw2)
    ypad = _sc_combine(z, ys, r["pos0"], r["pos1"])
    return jnp.concatenate([ypad[:, OUT:OUT + 1], ypad[:, :OUT]], axis=1)


# speedup vs baseline: 1.7431x; 1.0181x over previous
import functools

import jax
import jax.numpy as jnp
from jax import lax
from jax.experimental import pallas as pl
from jax.experimental.pallas import tpu as pltpu
from jax.experimental.pallas import tpu_sc as plsc

T = 2048
DIM = 2049
S = 1408
INTER = 1409
E = 64
K = 2

DPAD = 2176
OUT = 2048
YPAD = 2176
B = 64
NROWS_R = 4096 + E * B
NBLK_R = NROWS_R // B
BS = 256

NC = 2
NS = 16
NW = NC * NS
GCH = 16
CCH = 16

_f32 = jnp.float32
_i32 = jnp.int32


def _gate(x, gate_w):
    def body(x_ref, gw_ref, tw_ref, ti_ref):
        logits = jnp.dot(x_ref[...], gw_ref[...], preferred_element_type=_f32)
        s = jax.nn.softmax(logits, axis=-1)
        iota = lax.broadcasted_iota(_i32, (T, E), 1)
        m1 = jnp.max(s, axis=-1, keepdims=True)
        i1 = jnp.min(jnp.where(s >= m1, iota, E), axis=-1, keepdims=True)
        s2 = jnp.where(iota == i1, -1e30, s)
        m2 = jnp.max(s2, axis=-1, keepdims=True)
        i2 = jnp.min(jnp.where(s2 >= m2, iota, E), axis=-1, keepdims=True)
        tw_ref[...] = jnp.concatenate([m1, m2], axis=-1)
        ti_ref[...] = jnp.concatenate([i1, i2], axis=-1)

    return pl.pallas_call(
        body,
        out_shape=[jax.ShapeDtypeStruct((T, K), _f32),
                   jax.ShapeDtypeStruct((T, K), _i32)],
    )(x, gate_w)


def _routing(ti, tw):
    eid = ti.reshape(-1).astype(_i32)
    wts = tw.reshape(-1)
    order = jnp.argsort(eid, stable=True).astype(_i32)
    sorted_eid = eid[order]
    counts = jnp.zeros((E,), _i32).at[eid].add(1)
    blocks_e = (counts + B - 1) // B
    blk_start = jnp.concatenate(
        [jnp.zeros((1,), _i32), jnp.cumsum(blocks_e)[:-1].astype(_i32)])
    row_start = blk_start * B
    csum_counts = jnp.concatenate(
        [jnp.zeros((1,), _i32), jnp.cumsum(counts)[:-1].astype(_i32)])
    ranks = jnp.arange(T * K, dtype=_i32) - csum_counts[sorted_eid]
    dst_row = row_start[sorted_eid] + ranks
    src = jnp.zeros((NROWS_R,), _i32).at[dst_row].set(order // K)
    wvals = jnp.zeros((NROWS_R,), _f32).at[dst_row].set(wts[order])
    pos_flat = jnp.zeros((T * K,), _i32).at[order].set(dst_row)
    bi = jnp.arange(NBLK_R, dtype=_i32)
    blk_eid = jnp.clip(
        jnp.searchsorted(blk_start, bi, side="right").astype(_i32) - 1,
        0, E - 1)
    return dict(
        src=src,
        wrow2d=jnp.broadcast_to(wvals[:, None], (NROWS_R, 128)),
        pos0=pos_flat[0::2],
        pos1=pos_flat[1::2],
        blk_eid=blk_eid,
    )


def _mlp_stage1(xb, w1, w3):
    xb = xb.astype(w1.dtype)
    s1 = jnp.dot(xb, w1, preferred_element_type=_f32)
    s3 = jnp.dot(xb, w3, preferred_element_type=_f32)
    return jax.nn.silu(s1) * s3


def _mlp_stage2(sp, w2):
    t = jnp.sqrt(jnp.clip(
        jnp.sum(sp * sp, axis=-1, keepdims=True) + 1.0, 1e-6, None))
    hfull = jnp.concatenate([t, sp], axis=-1)
    s2 = jnp.dot(hfull.astype(w2.dtype), w2, preferred_element_type=_f32)
    t2 = jnp.sqrt(jnp.clip(
        jnp.sum(s2 * s2, axis=-1, keepdims=True) + 1.0, 1e-6, None))
    return s2, t2


def _wprep(w, d, s):
    wv = jnp.transpose(w, (1, 0, 2))
    nblk = (d + 7) // 8

    def body(in_ref, out_ref):
        out_ref[...] = jnp.transpose(in_ref[...], (1, 0, 2)).astype(
            jnp.bfloat16)

    return pl.pallas_call(
        body,
        grid=(nblk,),
        in_specs=[pl.BlockSpec((8, E, s), lambda b: (b, 0, 0))],
        out_specs=pl.BlockSpec((E, 8, s), lambda b: (0, b, 0)),
        out_shape=jax.ShapeDtypeStruct((E, d, s), jnp.bfloat16),
    )(wv)


def _expert_ab(blk_eid, xs, w1b, w3b, w2b, wrow2d):
    def body(eids_ref, x_ref, w1_ref, w3_ref, w2_ref, wr_ref, y_ref):
        sp = _mlp_stage1(x_ref[...][:, :DIM], w1_ref[0], w3_ref[0])
        s2, t2 = _mlp_stage2(sp, w2_ref[0])
        w = wr_ref[:, 0:1]
        z = jnp.zeros((B, YPAD - OUT - 1), _f32)
        y_ref[...] = jnp.concatenate([w * s2, w * t2, z], axis=-1)

    return pl.pallas_call(
        body,
        grid_spec=pltpu.PrefetchScalarGridSpec(
            num_scalar_prefetch=1,
            grid=(NBLK_R,),
            in_specs=[
                pl.BlockSpec((B, DPAD), lambda b, eids: (b, 0)),
                pl.BlockSpec((1, DIM, S), lambda b, eids: (eids[b], 0, 0)),
                pl.BlockSpec((1, DIM, S), lambda b, eids: (eids[b], 0, 0)),
                pl.BlockSpec((1, INTER, OUT), lambda b, eids: (eids[b], 0, 0)),
                pl.BlockSpec((B, 128), lambda b, eids: (b, 0)),
            ],
            out_specs=pl.BlockSpec((B, YPAD), lambda b, eids: (b, 0)),
        ),
        out_shape=jax.ShapeDtypeStruct((NROWS_R, YPAD), _f32),
    )(blk_eid, xs, w1b, w3b, w2b, wrow2d)


def _shared(x, sw1b, sw3b, sw2b):
    def body(x_ref, w1_ref, w3_ref, w2_ref, z_ref):
        sp = _mlp_stage1(x_ref[...], w1_ref[...], w3_ref[...])
        s2, t2 = _mlp_stage2(sp, w2_ref[...])
        z = jnp.zeros((BS, YPAD - OUT - 1), _f32)
        z_ref[...] = jnp.concatenate([s2, t2, z], axis=-1)

    return pl.pallas_call(
        body,
        grid=(T // BS,),
        in_specs=[
            pl.BlockSpec((BS, DIM), lambda b: (b, 0)),
            pl.BlockSpec((DIM, S), lambda b: (0, 0)),
            pl.BlockSpec((DIM, S), lambda b: (0, 0)),
            pl.BlockSpec((INTER, OUT), lambda b: (0, 0)),
        ],
        out_specs=pl.BlockSpec((BS, YPAD), lambda b: (b, 0)),
        out_shape=jax.ShapeDtypeStruct((T, YPAD), _f32),
    )(x, sw1b, sw3b, sw2b)


def _sc_mesh():
    return plsc.VectorSubcoreMesh(
        core_axis_name="c", subcore_axis_name="s", num_cores=NC)


def _sc_gather(src, xpad):
    nch = (NROWS_R // NW) // GCH

    @functools.partial(
        pl.kernel,
        out_type=jax.ShapeDtypeStruct((NROWS_R, DPAD), _f32),
        mesh=_sc_mesh(),
        scratch_types=[
            pltpu.VMEM((GCH,), _i32),
            pltpu.VMEM((GCH,), _i32),
            pltpu.VMEM((GCH, DPAD), _f32),
            pltpu.VMEM((GCH, DPAD), _f32),
            pltpu.SemaphoreType.DMA,
            pltpu.SemaphoreType.DMA,
            pltpu.SemaphoreType.DMA,
            pltpu.SemaphoreType.DMA,
        ],
    )
    def k(src_hbm, x_hbm, out_hbm, idx0, idx1, rows0, rows1, g0, g1, s0, s1):
        wid = lax.axis_index("s") * NC + lax.axis_index("c")
        base = wid * (NROWS_R // NW)
        idxs, rows, gsem, ssem = [idx0, idx1], [rows0, rows1], [g0, g1], [s0, s1]
        gh = [None, None]
        sh = [None, None]

        pltpu.sync_copy(src_hbm.at[pl.ds(base, GCH)], idxs[0])
        gh[0] = pltpu.async_copy(x_hbm.at[idxs[0]], rows[0], gsem[0])
        for i in range(1, nch):
            b, pb = i % 2, (i - 1) % 2
            if i >= 2:
                sh[b].wait()
            pltpu.sync_copy(src_hbm.at[pl.ds(base + i * GCH, GCH)], idxs[b])
            gh[b] = pltpu.async_copy(x_hbm.at[idxs[b]], rows[b], gsem[b])
            gh[pb].wait()
            sh[pb] = pltpu.async_copy(
                rows[pb], out_hbm.at[pl.ds(base + (i - 1) * GCH, GCH)],
                ssem[pb])
        lb = (nch - 1) % 2
        gh[lb].wait()
        sh[lb] = pltpu.async_copy(
            rows[lb], out_hbm.at[pl.ds(base + (nch - 1) * GCH, GCH)], ssem[lb])
        sh[lb].wait()
        if nch >= 2:
            sh[1 - lb].wait()

    return k(src, xpad)


def _sc_combine(z, ys, pos0, pos1):
    @functools.partial(
        pl.kernel,
        out_type=jax.ShapeDtypeStruct((T, YPAD), _f32),
        mesh=_sc_mesh(),
        scratch_types=[
            pltpu.VMEM((CCH,), _i32),
            pltpu.VMEM((CCH,), _i32),
            pltpu.VMEM((CCH, YPAD), _f32),
            pltpu.VMEM((CCH, YPAD), _f32),
            pltpu.VMEM((CCH, YPAD), _f32),
            pltpu.SemaphoreType.DMA,
        ],
    )
    def k(z_hbm, ys_hbm, pos0_hbm, pos1_hbm, out_hbm,
          i0_v, i1_v, rz_v, r0_v, r1_v, sem):
        wid = lax.axis_index("s") * NC + lax.axis_index("c")
        base = wid * (T // NW)

        def chunk(i, carry):
            off = base + i * CCH
            pltpu.sync_copy(pos0_hbm.at[pl.ds(off, CCH)], i0_v)
            pltpu.sync_copy(pos1_hbm.at[pl.ds(off, CCH)], i1_v)
            pltpu.sync_copy(z_hbm.at[pl.ds(off, CCH)], rz_v)
            pltpu.async_copy(ys_hbm.at[i0_v], r0_v, sem).wait()
            pltpu.async_copy(ys_hbm.at[i1_v], r1_v, sem).wait()

            def row(rr, c2):
                def col(cc, c3):
                    sl = pl.ds(cc * 16, 16)
                    rz_v[rr, sl] = rz_v[rr, sl] + r0_v[rr, sl] + r1_v[rr, sl]
                    return c3
                lax.fori_loop(0, YPAD // 16, col, 0)
                return c2

            lax.fori_loop(0, CCH, row, 0)
            pltpu.sync_copy(rz_v, out_hbm.at[pl.ds(off, CCH)])
            return carry

        lax.fori_loop(0, (T // NW) // CCH, chunk, 0)

    return k(z, ys, pos0, pos1)


def kernel(x, gate_w, w1, w2, w3, sw1, sw2, sw3):
    tw, ti = _gate(x, gate_w)
    r = _routing(ti, tw)
    xpad = jnp.pad(x, ((0, 0), (0, DPAD - DIM)))
    xs = _sc_gather(r["src"], xpad)
    w1b = _wprep(w1, DIM, S)
    w3b = _wprep(w3, DIM, S)
    w2b = _wprep(w2, INTER, OUT)
    ys = _expert_ab(r["blk_eid"], xs, w1b, w3b, w2b, r["wrow2d"])
    z = _shared(x, sw1.astype(jnp.bfloat16), sw3.astype(jnp.bfloat16),
                sw2.astype(jnp.bfloat16))
    ypad = _sc_combine(z, ys, r["pos0"], r["pos1"])
    return jnp.concatenate([ypad[:, OUT:OUT + 1], ypad[:, :OUT]], axis=1)


# speedup vs baseline: 1.8349x; 1.0527x over previous
import functools

import jax
import jax.numpy as jnp
from jax import lax
from jax.experimental import pallas as pl
from jax.experimental.pallas import tpu as pltpu
from jax.experimental.pallas import tpu_sc as plsc

T = 2048
DIM = 2049
S = 1408
INTER = 1409
E = 64
K = 2

DPAD = 2176
OUT = 2048
YPAD = 2176
B = 64
NROWS_R = 4096 + E * B
NBLK_R = NROWS_R // B
BS = 256

NC = 2
NS = 16
NW = NC * NS
GCH = 16
CCH = 16

_f32 = jnp.float32
_i32 = jnp.int32


def _gate(x, gate_w):
    def body(x_ref, gw_ref, tw_ref, ti_ref):
        logits = jnp.dot(x_ref[...], gw_ref[...], preferred_element_type=_f32)
        s = jax.nn.softmax(logits, axis=-1)
        iota = lax.broadcasted_iota(_i32, (T, E), 1)
        m1 = jnp.max(s, axis=-1, keepdims=True)
        i1 = jnp.min(jnp.where(s >= m1, iota, E), axis=-1, keepdims=True)
        s2 = jnp.where(iota == i1, -1e30, s)
        m2 = jnp.max(s2, axis=-1, keepdims=True)
        i2 = jnp.min(jnp.where(s2 >= m2, iota, E), axis=-1, keepdims=True)
        tw_ref[...] = jnp.concatenate([m1, m2], axis=-1)
        ti_ref[...] = jnp.concatenate([i1, i2], axis=-1)

    return pl.pallas_call(
        body,
        out_shape=[jax.ShapeDtypeStruct((T, K), _f32),
                   jax.ShapeDtypeStruct((T, K), _i32)],
    )(x, gate_w)


def _routing(ti, tw):
    eid = ti.reshape(-1).astype(_i32)
    wts = tw.reshape(-1)
    order = jnp.argsort(eid, stable=True).astype(_i32)
    sorted_eid = eid[order]
    counts = jnp.zeros((E,), _i32).at[eid].add(1)
    blocks_e = (counts + B - 1) // B
    blk_start = jnp.concatenate(
        [jnp.zeros((1,), _i32), jnp.cumsum(blocks_e)[:-1].astype(_i32)])
    row_start = blk_start * B
    csum_counts = jnp.concatenate(
        [jnp.zeros((1,), _i32), jnp.cumsum(counts)[:-1].astype(_i32)])
    ranks = jnp.arange(T * K, dtype=_i32) - csum_counts[sorted_eid]
    dst_row = row_start[sorted_eid] + ranks
    src = jnp.zeros((NROWS_R,), _i32).at[dst_row].set(order // K)
    wvals = jnp.zeros((NROWS_R,), _f32).at[dst_row].set(wts[order])
    pos_flat = jnp.zeros((T * K,), _i32).at[order].set(dst_row)
    bi = jnp.arange(NBLK_R, dtype=_i32)
    blk_eid = jnp.clip(
        jnp.searchsorted(blk_start, bi, side="right").astype(_i32) - 1,
        0, E - 1)
    return dict(
        src=src,
        wrow2d=jnp.broadcast_to(wvals[:, None], (NROWS_R, 128)),
        pos0=pos_flat[0::2],
        pos1=pos_flat[1::2],
        blk_eid=blk_eid,
    )


def _mlp_stage1(xb, w1, w3):
    xb = xb.astype(w1.dtype)
    s1 = jnp.dot(xb, w1, preferred_element_type=_f32)
    s3 = jnp.dot(xb, w3, preferred_element_type=_f32)
    return jax.nn.silu(s1) * s3


def _mlp_stage2(sp, w2):
    t = jnp.sqrt(jnp.clip(
        jnp.sum(sp * sp, axis=-1, keepdims=True) + 1.0, 1e-6, None))
    hfull = jnp.concatenate([t, sp], axis=-1)
    s2 = jnp.dot(hfull.astype(w2.dtype), w2, preferred_element_type=_f32)
    t2 = jnp.sqrt(jnp.clip(
        jnp.sum(s2 * s2, axis=-1, keepdims=True) + 1.0, 1e-6, None))
    return s2, t2


def _wprep(w, d, s):
    wv = jnp.transpose(w, (1, 0, 2))
    rows = 32
    nblk = (d + rows - 1) // rows

    def body(in_ref, out_ref):
        out_ref[...] = jnp.transpose(in_ref[...], (1, 0, 2)).astype(
            jnp.bfloat16)

    return pl.pallas_call(
        body,
        grid=(nblk,),
        in_specs=[pl.BlockSpec((rows, E, s), lambda b: (b, 0, 0))],
        out_specs=pl.BlockSpec((E, rows, s), lambda b: (0, b, 0)),
        out_shape=jax.ShapeDtypeStruct((E, d, s), jnp.bfloat16),
    )(wv)


def _expert_ab(blk_eid, xs, w1b, w3b, w2b, wrow2d):
    def body(eids_ref, x_ref, w1_ref, w3_ref, w2_ref, wr_ref, y_ref):
        sp = _mlp_stage1(x_ref[...][:, :DIM], w1_ref[0], w3_ref[0])
        s2, t2 = _mlp_stage2(sp, w2_ref[0])
        w = wr_ref[:, 0:1]
        z = jnp.zeros((B, YPAD - OUT - 1), _f32)
        y_ref[...] = jnp.concatenate([w * s2, w * t2, z], axis=-1)

    return pl.pallas_call(
        body,
        grid_spec=pltpu.PrefetchScalarGridSpec(
            num_scalar_prefetch=1,
            grid=(NBLK_R,),
            in_specs=[
                pl.BlockSpec((B, DPAD), lambda b, eids: (b, 0)),
                pl.BlockSpec((1, DIM, S), lambda b, eids: (eids[b], 0, 0)),
                pl.BlockSpec((1, DIM, S), lambda b, eids: (eids[b], 0, 0)),
                pl.BlockSpec((1, INTER, OUT), lambda b, eids: (eids[b], 0, 0)),
                pl.BlockSpec((B, 128), lambda b, eids: (b, 0)),
            ],
            out_specs=pl.BlockSpec((B, YPAD), lambda b, eids: (b, 0)),
        ),
        out_shape=jax.ShapeDtypeStruct((NROWS_R, YPAD), _f32),
    )(blk_eid, xs, w1b, w3b, w2b, wrow2d)


def _shared(x, sw1b, sw3b, sw2b):
    def body(x_ref, w1_ref, w3_ref, w2_ref, z_ref):
        sp = _mlp_stage1(x_ref[...], w1_ref[...], w3_ref[...])
        s2, t2 = _mlp_stage2(sp, w2_ref[...])
        z = jnp.zeros((BS, YPAD - OUT - 1), _f32)
        z_ref[...] = jnp.concatenate([s2, t2, z], axis=-1)

    return pl.pallas_call(
        body,
        grid=(T // BS,),
        in_specs=[
            pl.BlockSpec((BS, DIM), lambda b: (b, 0)),
            pl.BlockSpec((DIM, S), lambda b: (0, 0)),
            pl.BlockSpec((DIM, S), lambda b: (0, 0)),
            pl.BlockSpec((INTER, OUT), lambda b: (0, 0)),
        ],
        out_specs=pl.BlockSpec((BS, YPAD), lambda b: (b, 0)),
        out_shape=jax.ShapeDtypeStruct((T, YPAD), _f32),
    )(x, sw1b, sw3b, sw2b)


def _sc_mesh():
    return plsc.VectorSubcoreMesh(
        core_axis_name="c", subcore_axis_name="s", num_cores=NC)


def _sc_gather(src, xpad):
    nch = (NROWS_R // NW) // GCH

    @functools.partial(
        pl.kernel,
        out_type=jax.ShapeDtypeStruct((NROWS_R, DPAD), _f32),
        mesh=_sc_mesh(),
        scratch_types=[
            pltpu.VMEM((GCH,), _i32),
            pltpu.VMEM((GCH,), _i32),
            pltpu.VMEM((GCH, DPAD), _f32),
            pltpu.VMEM((GCH, DPAD), _f32),
            pltpu.SemaphoreType.DMA,
            pltpu.SemaphoreType.DMA,
            pltpu.SemaphoreType.DMA,
            pltpu.SemaphoreType.DMA,
        ],
    )
    def k(src_hbm, x_hbm, out_hbm, idx0, idx1, rows0, rows1, g0, g1, s0, s1):
        wid = lax.axis_index("s") * NC + lax.axis_index("c")
        base = wid * (NROWS_R // NW)
        idxs, rows, gsem, ssem = [idx0, idx1], [rows0, rows1], [g0, g1], [s0, s1]
        gh = [None, None]
        sh = [None, None]

        pltpu.sync_copy(src_hbm.at[pl.ds(base, GCH)], idxs[0])
        gh[0] = pltpu.async_copy(x_hbm.at[idxs[0]], rows[0], gsem[0])
        for i in range(1, nch):
            b, pb = i % 2, (i - 1) % 2
            if i >= 2:
                sh[b].wait()
            pltpu.sync_copy(src_hbm.at[pl.ds(base + i * GCH, GCH)], idxs[b])
            gh[b] = pltpu.async_copy(x_hbm.at[idxs[b]], rows[b], gsem[b])
            gh[pb].wait()
            sh[pb] = pltpu.async_copy(
                rows[pb], out_hbm.at[pl.ds(base + (i - 1) * GCH, GCH)],
                ssem[pb])
        lb = (nch - 1) % 2
        gh[lb].wait()
        sh[lb] = pltpu.async_copy(
            rows[lb], out_hbm.at[pl.ds(base + (nch - 1) * GCH, GCH)], ssem[lb])
        sh[lb].wait()
        if nch >= 2:
            sh[1 - lb].wait()

    return k(src, xpad)


def _sc_combine(z, ys, pos0, pos1):
    @functools.partial(
        pl.kernel,
        out_type=jax.ShapeDtypeStruct((T, YPAD), _f32),
        mesh=_sc_mesh(),
        scratch_types=[
            pltpu.VMEM((CCH,), _i32),
            pltpu.VMEM((CCH,), _i32),
            pltpu.VMEM((CCH, YPAD), _f32),
            pltpu.VMEM((CCH, YPAD), _f32),
            pltpu.VMEM((CCH, YPAD), _f32),
            pltpu.SemaphoreType.DMA,
        ],
    )
    def k(z_hbm, ys_hbm, pos0_hbm, pos1_hbm, out_hbm,
          i0_v, i1_v, rz_v, r0_v, r1_v, sem):
        wid = lax.axis_index("s") * NC + lax.axis_index("c")
        base = wid * (T // NW)

        def chunk(i, carry):
            off = base + i * CCH
            pltpu.sync_copy(pos0_hbm.at[pl.ds(off, CCH)], i0_v)
            pltpu.sync_copy(pos1_hbm.at[pl.ds(off, CCH)], i1_v)
            pltpu.sync_copy(z_hbm.at[pl.ds(off, CCH)], rz_v)
            pltpu.async_copy(ys_hbm.at[i0_v], r0_v, sem).wait()
            pltpu.async_copy(ys_hbm.at[i1_v], r1_v, sem).wait()

            def row(rr, c2):
                def col(cc, c3):
                    sl = pl.ds(cc * 16, 16)
                    rz_v[rr, sl] = rz_v[rr, sl] + r0_v[rr, sl] + r1_v[rr, sl]
                    return c3
                lax.fori_loop(0, YPAD // 16, col, 0)
                return c2

            lax.fori_loop(0, CCH, row, 0)
            pltpu.sync_copy(rz_v, out_hbm.at[pl.ds(off, CCH)])
            return carry

        lax.fori_loop(0, (T // NW) // CCH, chunk, 0)

    return k(z, ys, pos0, pos1)


def kernel(x, gate_w, w1, w2, w3, sw1, sw2, sw3):
    tw, ti = _gate(x, gate_w)
    r = _routing(ti, tw)
    xpad = jnp.pad(x, ((0, 0), (0, DPAD - DIM)))
    xs = _sc_gather(r["src"], xpad)
    w1b = _wprep(w1, DIM, S)
    w3b = _wprep(w3, DIM, S)
    w2b = _wprep(w2, INTER, OUT)
    ys = _expert_ab(r["blk_eid"], xs, w1b, w3b, w2b, r["wrow2d"])
    z = _shared(x, sw1.astype(jnp.bfloat16), sw3.astype(jnp.bfloat16),
                sw2.astype(jnp.bfloat16))
    ypad = _sc_combine(z, ys, r["pos0"], r["pos1"])
    return jnp.concatenate([ypad[:, OUT:OUT + 1], ypad[:, :OUT]], axis=1)
